# Initial kernel scaffold; baseline (speedup 1.0000x reference)
import jax
import jax.numpy as jnp
from jax.experimental import pallas as pl


def kernel(x, edge_index, edge_attr, W_e, b_e, W1, b1, W2, b2, gamma, beta):
    raise NotImplementedError("write your pallas kernel here")


# speedup vs baseline: 2.0463x; 2.0463x over previous
import functools

import jax
import jax.numpy as jnp
from jax import lax
from jax.experimental import pallas as pl
from jax.experimental.pallas import tpu as pltpu
from jax.experimental.pallas import tpu_sc as plsc

N = 50000
E = 800000
H = 64
HH = 32
NLAYERS = 4
BN_EPS = 1e-5

CHUNK = 128
ROWS = E // CHUNK
NSUB = 16
ROWS_PER_TILE = -(-ROWS // NSUB)
ZROWS = 200
NWB = N // ZROWS

BM = 2000
NB = N // BM
BE = 5000
NEB = E // BE


def _bf16(v):
    return v.astype(jnp.bfloat16).astype(jnp.float32)


def _t0_body(attr_ref, we_ref, be_ref, e0_ref, e1_ref):
    a = _bf16(attr_ref[...])
    W = _bf16(we_ref[...])
    e = be_ref[...] + (a[:, 0:1] * W[0:1, :] + a[:, 1:2] * W[1:2, :]
                       + a[:, 2:3] * W[2:3, :] + a[:, 3:4] * W[3:4, :])
    e0_ref[...] = e[:, :HH]
    e1_ref[...] = e[:, HH:]


def _edge_transform(edge_attr, W_e, b_e):
    return pl.pallas_call(
        _t0_body,
        grid=(NEB,),
        in_specs=[
            pl.BlockSpec((BE, 4), lambda i: (i, 0)),
            pl.BlockSpec((4, H), lambda i: (0, 0)),
            pl.BlockSpec((1, H), lambda i: (0, 0)),
        ],
        out_specs=[
            pl.BlockSpec((BE, HH), lambda i: (i, 0)),
            pl.BlockSpec((BE, HH), lambda i: (i, 0)),
        ],
        out_shape=[
            jax.ShapeDtypeStruct((E, HH), jnp.float32),
            jax.ShapeDtypeStruct((E, HH), jnp.float32),
        ],
    )(edge_attr, W_e, b_e.reshape(1, H))


def _sc_body(h0_hbm, h1_hbm, e0_hbm, e1_hbm, src_hbm, dst_hbm,
             a0_hbm, a1_hbm, acc, sidx, didx, ebuf, hbuf, zbuf, sem):
    c = lax.axis_index("c")
    s = lax.axis_index("s")

    def per_core(h_hbm, e_hbm, a_hbm):
        z = jnp.zeros((16,), jnp.float32)
        for i in range(ZROWS):
            zbuf[i, pl.ds(0, 16)] = z
            zbuf[i, pl.ds(16, 16)] = z
        for k in range(-(-NWB // NSUB)):
            cid = k * NSUB + s

            @pl.when(cid < NWB)
            def _():
                pltpu.sync_copy(zbuf, acc.at[pl.ds(cid * ZROWS, ZROWS)])
        plsc.subcore_barrier()

        @pl.loop(0, ROWS_PER_TILE)
        def _main(k):
            r = k * NSUB + s

            @pl.when(r < ROWS)
            def _():
                base = r * CHUNK
                pltpu.sync_copy(src_hbm.at[pl.ds(base, CHUNK)], sidx)
                pltpu.sync_copy(dst_hbm.at[pl.ds(base, CHUNK)], didx.at[0])
                pltpu.sync_copy(e_hbm.at[pl.ds(base, CHUNK)], ebuf)
                pltpu.async_copy(h_hbm.at[sidx], hbuf, sem).wait()
                for i in range(CHUNK):
                    for j in (0, 16):
                        hv = hbuf[i, pl.ds(j, 16)]
                        ev = ebuf[i, pl.ds(j, 16)]
                        hbuf[i, pl.ds(j, 16)] = jnp.maximum(hv + ev, 0.0)
                pltpu.sync_copy(hbuf, acc.at[didx.at[0]], add=True)

        plsc.subcore_barrier()
        for k in range(-(-NWB // NSUB)):
            cid = k * NSUB + s

            @pl.when(cid < NWB)
            def _():
                off = cid * ZROWS
                pltpu.sync_copy(acc.at[pl.ds(off, ZROWS)],
                                a_hbm.at[pl.ds(off, ZROWS)])

    @pl.when(c == 0)
    def _():
        per_core(h0_hbm, e0_hbm, a0_hbm)

    @pl.when(c == 1)
    def _():
        per_core(h1_hbm, e1_hbm, a1_hbm)


def _aggregate(h0, h1, e0, e1, src, dst):
    mesh = plsc.VectorSubcoreMesh(core_axis_name="c", subcore_axis_name="s")
    f = pl.kernel(
        _sc_body,
        mesh=mesh,
        compiler_params=pltpu.CompilerParams(use_tc_tiling_on_sc=False),
        out_type=(
            jax.ShapeDtypeStruct((N, HH), jnp.float32),
            jax.ShapeDtypeStruct((N, HH), jnp.float32),
        ),
        scratch_types=[
            pltpu.VMEM_SHARED((N, HH), jnp.float32),
            pltpu.VMEM((CHUNK,), jnp.int32),
            pltpu.VMEM((1, CHUNK), jnp.int32),
            pltpu.VMEM((CHUNK, HH), jnp.float32),
            pltpu.VMEM((CHUNK, HH), jnp.float32),
            pltpu.VMEM((ZROWS, HH), jnp.float32),
            pltpu.SemaphoreType.DMA,
        ],
    )
    return f(h0, h1, e0, e1, src, dst)


def _t1_body(h_ref, a0_ref, a1_ref, w1_ref, b1_ref, w2_ref, b2_ref,
             out_ref, ps_ref, pq_ref):
    pre = h_ref[...] + jnp.concatenate([a0_ref[...], a1_ref[...]], axis=1)
    hid = jnp.maximum(
        jnp.dot(pre.astype(jnp.bfloat16), w1_ref[...].astype(jnp.bfloat16),
                preferred_element_type=jnp.float32)
        + b1_ref[...], 0.0)
    out = (jnp.dot(hid.astype(jnp.bfloat16), w2_ref[...].astype(jnp.bfloat16),
                   preferred_element_type=jnp.float32)
           + b2_ref[...])
    out_ref[...] = out
    ps_ref[...] = jnp.sum(out, axis=0).reshape(1, 1, H)
    pq_ref[...] = jnp.sum(out * out, axis=0).reshape(1, 1, H)


def _dense1(h, a0, a1, W1l, b1l, W2l, b2l):
    return pl.pallas_call(
        _t1_body,
        grid=(NB,),
        in_specs=[
            pl.BlockSpec((BM, H), lambda i: (i, 0)),
            pl.BlockSpec((BM, HH), lambda i: (i, 0)),
            pl.BlockSpec((BM, HH), lambda i: (i, 0)),
            pl.BlockSpec((H, H), lambda i: (0, 0)),
            pl.BlockSpec((1, H), lambda i: (0, 0)),
            pl.BlockSpec((H, H), lambda i: (0, 0)),
            pl.BlockSpec((1, H), lambda i: (0, 0)),
        ],
        out_specs=[
            pl.BlockSpec((BM, H), lambda i: (i, 0)),
            pl.BlockSpec((1, 1, H), lambda i: (i, 0, 0)),
            pl.BlockSpec((1, 1, H), lambda i: (i, 0, 0)),
        ],
        out_shape=[
            jax.ShapeDtypeStruct((N, H), jnp.float32),
            jax.ShapeDtypeStruct((NB, 1, H), jnp.float32),
            jax.ShapeDtypeStruct((NB, 1, H), jnp.float32),
        ],
    )(h, a0, a1, W1l, b1l, W2l, b2l)


def _t2_body(out_ref, ps_ref, pq_ref, hprev_ref, g_ref, bt_ref,
             h_ref, h0_ref, h1_ref):
    mu = jnp.sum(ps_ref[...], axis=(0, 1)) / N
    var = jnp.sum(pq_ref[...], axis=(0, 1)) / N - mu * mu
    inv = lax.rsqrt(var + BN_EPS)
    y = (out_ref[...] - mu) * (inv * g_ref[...]) + bt_ref[...]
    hn = jnp.maximum(y, 0.0) + hprev_ref[...]
    h_ref[...] = hn
    h0_ref[...] = hn[:, :HH]
    h1_ref[...] = hn[:, HH:]


def _dense2(out, ps, pq, hprev, gl, btl):
    return pl.pallas_call(
        _t2_body,
        grid=(NB,),
        in_specs=[
            pl.BlockSpec((BM, H), lambda i: (i, 0)),
            pl.BlockSpec((NB, 1, H), lambda i: (0, 0, 0)),
            pl.BlockSpec((NB, 1, H), lambda i: (0, 0, 0)),
            pl.BlockSpec((BM, H), lambda i: (i, 0)),
            pl.BlockSpec((1, H), lambda i: (0, 0)),
            pl.BlockSpec((1, H), lambda i: (0, 0)),
        ],
        out_specs=[
            pl.BlockSpec((BM, H), lambda i: (i, 0)),
            pl.BlockSpec((BM, HH), lambda i: (i, 0)),
            pl.BlockSpec((BM, HH), lambda i: (i, 0)),
        ],
        out_shape=[
            jax.ShapeDtypeStruct((N, H), jnp.float32),
            jax.ShapeDtypeStruct((N, HH), jnp.float32),
            jax.ShapeDtypeStruct((N, HH), jnp.float32),
        ],
    )(out, ps, pq, hprev, gl, btl)


def kernel(x, edge_index, edge_attr, W_e, b_e, W1, b1, W2, b2, gamma, beta):
    src = edge_index[0]
    dst = edge_index[1]
    e0, e1 = _edge_transform(edge_attr, W_e, b_e)
    h = x
    h0 = x[:, :HH]
    h1 = x[:, HH:]
    for i in range(NLAYERS):
        a0, a1 = _aggregate(h0, h1, e0, e1, src, dst)
        out, ps, pq = _dense1(h, a0, a1, W1[i], b1[i].reshape(1, H),
                              W2[i], b2[i].reshape(1, H))
        h, h0, h1 = _dense2(out, ps, pq, h, gamma[i].reshape(1, H),
                            beta[i].reshape(1, H))
    return h


# speedup vs baseline: 4.2264x; 2.0653x over previous
import functools

import jax
import jax.numpy as jnp
from jax import lax
from jax.experimental import pallas as pl
from jax.experimental.pallas import tpu as pltpu
from jax.experimental.pallas import tpu_sc as plsc

N = 50000
E = 800000
H = 64
HH = 32
NLAYERS = 4
BN_EPS = 1e-5

CHUNK = 80
ROWS = E // CHUNK
NSUB = 16
ROWS_PER_TILE = -(-ROWS // NSUB)
ZROWS = 40
NWB = N // ZROWS

BM = 2000
NB = N // BM
BE = 5000
NEB = E // BE


def _bf16(v):
    return v.astype(jnp.bfloat16).astype(jnp.float32)


def _t0_body(attr_ref, we_ref, be_ref, e0_ref, e1_ref):
    a = _bf16(attr_ref[...])
    W = _bf16(we_ref[...])
    e = be_ref[...] + (a[:, 0:1] * W[0:1, :] + a[:, 1:2] * W[1:2, :]
                       + a[:, 2:3] * W[2:3, :] + a[:, 3:4] * W[3:4, :])
    e0_ref[...] = e[:, :HH]
    e1_ref[...] = e[:, HH:]


def _edge_transform(edge_attr, W_e, b_e):
    return pl.pallas_call(
        _t0_body,
        grid=(NEB,),
        in_specs=[
            pl.BlockSpec((BE, 4), lambda i: (i, 0)),
            pl.BlockSpec((4, H), lambda i: (0, 0)),
            pl.BlockSpec((1, H), lambda i: (0, 0)),
        ],
        out_specs=[
            pl.BlockSpec((BE, HH), lambda i: (i, 0)),
            pl.BlockSpec((BE, HH), lambda i: (i, 0)),
        ],
        out_shape=[
            jax.ShapeDtypeStruct((E, HH), jnp.float32),
            jax.ShapeDtypeStruct((E, HH), jnp.float32),
        ],
    )(edge_attr, W_e, b_e.reshape(1, H))


NBUF = 4


def _sc_body(h0_hbm, h1_hbm, e0_hbm, e1_hbm, src_hbm, dst_hbm,
             a0_hbm, a1_hbm, acc, sidx, didx, ebuf, hbuf, zbuf,
             isem, gsem, esem, ssem):
    c = lax.axis_index("c")
    s = lax.axis_index("s")
    nvalid = jnp.where(s < ROWS - (ROWS_PER_TILE - 1) * NSUB,
                       ROWS_PER_TILE, ROWS_PER_TILE - 1)

    def per_core(h_hbm, e_hbm, a_hbm):
        z = jnp.zeros((16,), jnp.float32)
        for i in range(ZROWS):
            zbuf[i, pl.ds(0, 16)] = z
            zbuf[i, pl.ds(16, 16)] = z
        for k in range(-(-NWB // NSUB)):
            cid = k * NSUB + s

            @pl.when(cid < NWB)
            def _():
                pltpu.sync_copy(zbuf, acc.at[pl.ds(cid * ZROWS, ZROWS)])
        plsc.subcore_barrier()

        def idx_issue(j, b):
            @pl.when(j < nvalid)
            def _():
                base = (j * NSUB + s) * CHUNK
                pltpu.async_copy(src_hbm.at[pl.ds(base, CHUNK)],
                                 sidx.at[b], isem.at[b])
                pltpu.async_copy(dst_hbm.at[pl.ds(base, CHUNK)],
                                 didx.at[b], isem.at[b])

        def ge_issue(j, b):
            @pl.when(j < nvalid)
            def _():
                base = (j * NSUB + s) * CHUNK
                pltpu.make_async_copy(src_hbm.at[pl.ds(base, CHUNK)],
                                      sidx.at[b], isem.at[b]).wait()
                pltpu.make_async_copy(dst_hbm.at[pl.ds(base, CHUNK)],
                                      didx.at[b], isem.at[b]).wait()
                pltpu.async_copy(h_hbm.at[sidx.at[b]], hbuf.at[b], gsem.at[b])
                pltpu.async_copy(e_hbm.at[pl.ds(base, CHUNK)],
                                 ebuf.at[b], esem.at[b])

        def scatter_drain(j, b):
            @pl.when((0 <= j) & (j < nvalid))
            def _():
                pltpu.make_async_copy(hbuf.at[b], acc.at[didx.at[b]],
                                      ssem.at[b]).wait()

        def compute(j, b):
            @pl.when(j < nvalid)
            def _():
                base = (j * NSUB + s) * CHUNK
                pltpu.make_async_copy(h_hbm.at[sidx.at[b]], hbuf.at[b],
                                      gsem.at[b]).wait()
                pltpu.make_async_copy(e_hbm.at[pl.ds(base, CHUNK)],
                                      ebuf.at[b], esem.at[b]).wait()

                @pl.loop(0, CHUNK, step=8)
                def _relu(i):
                    for di in range(8):
                        for jj in (0, 16):
                            hv = hbuf[b, i + di, pl.ds(jj, 16)]
                            ev = ebuf[b, i + di, pl.ds(jj, 16)]
                            hbuf[b, i + di, pl.ds(jj, 16)] = jnp.maximum(
                                hv + ev, 0.0)
                pltpu.async_copy(hbuf.at[b], acc.at[didx.at[b]], ssem.at[b],
                                 add=True)

        idx_issue(jnp.int32(0), 0)
        idx_issue(jnp.int32(1), 1)
        ge_issue(jnp.int32(0), 0)

        @pl.loop(0, ROWS_PER_TILE + 2, step=NBUF)
        def _main(k):
            for u in range(NBUF):
                kk = k + u
                scatter_drain(kk - 2, (u + 2) % NBUF)
                idx_issue(kk + 2, (u + 2) % NBUF)
                ge_issue(kk + 1, (u + 1) % NBUF)
                compute(kk, u)

        plsc.subcore_barrier()
        for k in range(-(-NWB // NSUB)):
            cid = k * NSUB + s

            @pl.when(cid < NWB)
            def _():
                off = cid * ZROWS
                pltpu.sync_copy(acc.at[pl.ds(off, ZROWS)],
                                a_hbm.at[pl.ds(off, ZROWS)])

    @pl.when(c == 0)
    def _():
        per_core(h0_hbm, e0_hbm, a0_hbm)

    @pl.when(c == 1)
    def _():
        per_core(h1_hbm, e1_hbm, a1_hbm)


def _aggregate(h0, h1, e0, e1, src, dst):
    mesh = plsc.VectorSubcoreMesh(core_axis_name="c", subcore_axis_name="s")
    f = pl.kernel(
        _sc_body,
        mesh=mesh,
        compiler_params=pltpu.CompilerParams(use_tc_tiling_on_sc=False),
        out_type=(
            jax.ShapeDtypeStruct((N, HH), jnp.float32),
            jax.ShapeDtypeStruct((N, HH), jnp.float32),
        ),
        scratch_types=[
            pltpu.VMEM_SHARED((N, HH), jnp.float32),
            pltpu.VMEM((NBUF, CHUNK), jnp.int32),
            pltpu.VMEM((NBUF, CHUNK), jnp.int32),
            pltpu.VMEM((NBUF, CHUNK, HH), jnp.float32),
            pltpu.VMEM((NBUF, CHUNK, HH), jnp.float32),
            pltpu.VMEM((ZROWS, HH), jnp.float32),
            pltpu.SemaphoreType.DMA((NBUF,)),
            pltpu.SemaphoreType.DMA((NBUF,)),
            pltpu.SemaphoreType.DMA((NBUF,)),
            pltpu.SemaphoreType.DMA((NBUF,)),
        ],
    )
    return f(h0, h1, e0, e1, src, dst)


def _t1_body(h_ref, a0_ref, a1_ref, w1_ref, b1_ref, w2_ref, b2_ref,
             out_ref, ps_ref, pq_ref):
    pre = h_ref[...] + jnp.concatenate([a0_ref[...], a1_ref[...]], axis=1)
    hid = jnp.maximum(
        jnp.dot(pre.astype(jnp.bfloat16), w1_ref[...].astype(jnp.bfloat16),
                preferred_element_type=jnp.float32)
        + b1_ref[...], 0.0)
    out = (jnp.dot(hid.astype(jnp.bfloat16), w2_ref[...].astype(jnp.bfloat16),
                   preferred_element_type=jnp.float32)
           + b2_ref[...])
    out_ref[...] = out
    ps_ref[...] = jnp.sum(out, axis=0).reshape(1, 1, H)
    pq_ref[...] = jnp.sum(out * out, axis=0).reshape(1, 1, H)


def _dense1(h, a0, a1, W1l, b1l, W2l, b2l):
    return pl.pallas_call(
        _t1_body,
        grid=(NB,),
        in_specs=[
            pl.BlockSpec((BM, H), lambda i: (i, 0)),
            pl.BlockSpec((BM, HH), lambda i: (i, 0)),
            pl.BlockSpec((BM, HH), lambda i: (i, 0)),
            pl.BlockSpec((H, H), lambda i: (0, 0)),
            pl.BlockSpec((1, H), lambda i: (0, 0)),
            pl.BlockSpec((H, H), lambda i: (0, 0)),
            pl.BlockSpec((1, H), lambda i: (0, 0)),
        ],
        out_specs=[
            pl.BlockSpec((BM, H), lambda i: (i, 0)),
            pl.BlockSpec((1, 1, H), lambda i: (i, 0, 0)),
            pl.BlockSpec((1, 1, H), lambda i: (i, 0, 0)),
        ],
        out_shape=[
            jax.ShapeDtypeStruct((N, H), jnp.float32),
            jax.ShapeDtypeStruct((NB, 1, H), jnp.float32),
            jax.ShapeDtypeStruct((NB, 1, H), jnp.float32),
        ],
    )(h, a0, a1, W1l, b1l, W2l, b2l)


def _t2_body(out_ref, ps_ref, pq_ref, hprev_ref, g_ref, bt_ref,
             h_ref, h0_ref, h1_ref):
    mu = jnp.sum(ps_ref[...], axis=(0, 1)) / N
    var = jnp.sum(pq_ref[...], axis=(0, 1)) / N - mu * mu
    inv = lax.rsqrt(var + BN_EPS)
    y = (out_ref[...] - mu) * (inv * g_ref[...]) + bt_ref[...]
    hn = jnp.maximum(y, 0.0) + hprev_ref[...]
    h_ref[...] = hn
    h0_ref[...] = hn[:, :HH]
    h1_ref[...] = hn[:, HH:]


def _dense2(out, ps, pq, hprev, gl, btl):
    return pl.pallas_call(
        _t2_body,
        grid=(NB,),
        in_specs=[
            pl.BlockSpec((BM, H), lambda i: (i, 0)),
            pl.BlockSpec((NB, 1, H), lambda i: (0, 0, 0)),
            pl.BlockSpec((NB, 1, H), lambda i: (0, 0, 0)),
            pl.BlockSpec((BM, H), lambda i: (i, 0)),
            pl.BlockSpec((1, H), lambda i: (0, 0)),
            pl.BlockSpec((1, H), lambda i: (0, 0)),
        ],
        out_specs=[
            pl.BlockSpec((BM, H), lambda i: (i, 0)),
            pl.BlockSpec((BM, HH), lambda i: (i, 0)),
            pl.BlockSpec((BM, HH), lambda i: (i, 0)),
        ],
        out_shape=[
            jax.ShapeDtypeStruct((N, H), jnp.float32),
            jax.ShapeDtypeStruct((N, HH), jnp.float32),
            jax.ShapeDtypeStruct((N, HH), jnp.float32),
        ],
    )(out, ps, pq, hprev, gl, btl)


def kernel(x, edge_index, edge_attr, W_e, b_e, W1, b1, W2, b2, gamma, beta):
    src = edge_index[0]
    dst = edge_index[1]
    e0, e1 = _edge_transform(edge_attr, W_e, b_e)
    h = x
    h0 = x[:, :HH]
    h1 = x[:, HH:]
    for i in range(NLAYERS):
        a0, a1 = _aggregate(h0, h1, e0, e1, src, dst)
        out, ps, pq = _dense1(h, a0, a1, W1[i], b1[i].reshape(1, H),
                              W2[i], b2[i].reshape(1, H))
        h, h0, h1 = _dense2(out, ps, pq, h, gamma[i].reshape(1, H),
                            beta[i].reshape(1, H))
    return h
